# baseline (speedup 1.0000x reference)
import jax
import jax.numpy as jnp
from jax import lax
from jax.experimental import pallas as pl
from jax.experimental.pallas import tpu as pltpu, tpu_sc as plsc

_CHUNK = 64


def _sc_copy(seq_len, d_model, pe):
    info = plsc.get_sparse_core_info()
    nw = info.num_cores * info.num_subcores
    rows_per_w = seq_len // nw
    n_chunks = rows_per_w // _CHUNK
    mesh = plsc.VectorSubcoreMesh(core_axis_name="c", subcore_axis_name="s")

    def body(pe_hbm, out_hbm, buf_v):
        wid = lax.axis_index("s") * info.num_cores + lax.axis_index("c")
        base = wid * rows_per_w

        for k in range(n_chunks):
            off = base + k * _CHUNK
            pltpu.sync_copy(pe_hbm.at[pl.ds(off, _CHUNK)], buf_v)
            pltpu.sync_copy(buf_v, out_hbm.at[pl.ds(off, _CHUNK)])

    return pl.kernel(
        body,
        mesh=mesh,
        out_type=jax.ShapeDtypeStruct((seq_len, d_model), jnp.float32),
        scratch_types=[pltpu.VMEM((_CHUNK, d_model), jnp.float32)],
    )(pe)


def kernel(x, pe):
    seq_len = x.shape[-1]
    d_model = pe.shape[-1]
    out = _sc_copy(seq_len, d_model, pe)
    return out.reshape(1, seq_len, d_model)


# speedup vs baseline: 4.0445x; 4.0445x over previous
import math

import jax
import jax.numpy as jnp
from jax.experimental import pallas as pl
from jax.experimental.pallas import tpu as pltpu

_LOG1E4 = math.log(10000.0)
_HALF_PI = math.pi / 2.0
_H = 64
_SEED = 8
_HI_PER_STEP = 16


def _seed_and_expand(s_ref, c_ref, seed_angle, sk, ck):
    s_ref[0:_SEED, :] = jnp.sin(seed_angle)
    c_ref[0:_SEED, :] = jnp.sin(seed_angle + _HALF_PI)
    n = _SEED
    for _ in range(3):
        s0 = s_ref[0:n, :]
        c0 = c_ref[0:n, :]
        s_ref[n : 2 * n, :] = s0 * ck + c0 * sk
        c_ref[n : 2 * n, :] = c0 * ck - s0 * sk
        sk, ck = 2.0 * sk * ck, ck * ck - sk * sk
        n *= 2


def _make_body(d_model):
    def body(o_ref, s1_ref, c1_ref, s2_ref, c2_ref):
        i = pl.program_id(0)

        @pl.when(i == 0)
        def _init():
            col = jax.lax.broadcasted_iota(jnp.int32, (1, d_model), 1)
            parity = col % 2
            k2 = (col - parity).astype(jnp.float32)
            freq = jnp.exp(k2 * (-_LOG1E4 / d_model))
            phase = parity.astype(jnp.float32) * _HALF_PI
            r = jax.lax.broadcasted_iota(jnp.int32, (_SEED, d_model), 0)
            rf = r.astype(jnp.float32)
            _seed_and_expand(
                s1_ref, c1_ref, (rf + 1.0) * freq + phase,
                jnp.sin(freq * float(_SEED)),
                jnp.sin(freq * float(_SEED) + _HALF_PI),
            )
            f64 = freq * float(_H)
            _seed_and_expand(
                s2_ref, c2_ref, rf * f64,
                jnp.sin(f64 * float(_SEED)),
                jnp.sin(f64 * float(_SEED) + _HALF_PI),
            )

        s2 = s2_ref[pl.ds(i * _HI_PER_STEP, _HI_PER_STEP), :][:, None, :]
        c2 = c2_ref[pl.ds(i * _HI_PER_STEP, _HI_PER_STEP), :][:, None, :]
        s1 = s1_ref[...][None]
        c1 = c1_ref[...][None]
        blk = s2 * c1 + c2 * s1
        o_ref[...] = blk.reshape(_HI_PER_STEP * _H, d_model)

    return body


def kernel(x, pe):
    seq_len = x.shape[-1]
    d_model = pe.shape[-1]
    rows_per_step = _HI_PER_STEP * _H
    grid = (seq_len // rows_per_step,)
    scratch = [pltpu.VMEM((_H, d_model), jnp.float32) for _ in range(4)]
    out2 = pl.pallas_call(
        _make_body(d_model),
        grid=grid,
        out_specs=pl.BlockSpec((rows_per_step, d_model), lambda i: (i, 0)),
        out_shape=jax.ShapeDtypeStruct((seq_len, d_model), pe.dtype),
        scratch_shapes=scratch,
    )()
    return out2[None]
